# baseline (speedup 1.0000x reference)
import functools

import jax
import jax.numpy as jnp
from jax import lax
from jax.experimental import pallas as pl
from jax.experimental.pallas import tpu as pltpu
from jax.experimental.pallas import tpu_sc as plsc

N = 10000
TOTAL = N * N
K = N * 16
NB_SHIFT = 16
NBUCKET = 16384
CAP = 32768
NT = 32

_MESH = plsc.VectorSubcoreMesh(core_axis_name="c", subcore_axis_name="s")


def _row_start(wid):
    return jnp.where(wid < 16, wid * 313, 16 * 313 + (wid - 16) * 312)


@functools.partial(
    pl.kernel,
    out_type=jax.ShapeDtypeStruct((NT, NBUCKET), jnp.int32),
    mesh=_MESH,
    compiler_params=pltpu.CompilerParams(needs_layout_passes=False),
    scratch_types=[
        pltpu.VMEM((N,), jnp.int32),
        pltpu.VMEM((N,), jnp.int32),
        pltpu.VMEM((NBUCKET,), jnp.int32),
        pltpu.SemaphoreType.DMA,
        pltpu.SemaphoreType.DMA,
    ],
)
def _sc_hist(x_hbm, hist_out, bufa, bufb, histbuf, sema, semb):
    wid = lax.axis_index("s") * 2 + lax.axis_index("c")
    start = _row_start(wid)
    nrows = jnp.where(wid < 16, 313, 312)
    end = start + nrows
    zeros = jnp.zeros((16,), jnp.int32)

    def zbody(i, carry):
        for u in range(8):
            histbuf[pl.ds((i * 8 + u) * 16, 16)] = zeros
        return carry

    lax.fori_loop(0, NBUCKET // 128, zbody, 0)

    ones = jnp.ones((16,), jnp.int32)

    def issue(buf, sem, r):
        @pl.when(r < end)
        def _():
            pltpu.async_copy(x_hbm.at[r], buf, sem)

    def wait(buf, sem, r):
        pltpu.make_async_copy(x_hbm.at[r], buf, sem).wait()

    def process(buf):
        @plsc.parallel_loop(0, N // 16, unroll=12)
        def _(i):
            kb = buf[pl.ds(i * 16, 16)]
            plsc.addupdate_scatter(histbuf, [kb >> NB_SHIFT], ones)

    issue(bufa, sema, start)
    issue(bufb, semb, start + 1)

    def pair_body(i, carry):
        r = start + 2 * i
        wait(bufa, sema, r)
        process(bufa)
        issue(bufa, sema, r + 2)
        wait(bufb, semb, r + 1)
        process(bufb)
        issue(bufb, semb, r + 3)
        return carry

    lax.fori_loop(0, nrows // 2, pair_body, 0)

    @pl.when(nrows % 2 == 1)
    def _():
        wait(bufa, sema, end - 1)
        process(bufa)

    pltpu.sync_copy(histbuf, hist_out.at[wid])


def _findbucket_body(hist_ref, out_ref):
    h = hist_ref[...]
    cs = jnp.sum(h, axis=0, keepdims=True)
    iota = lax.broadcasted_iota(jnp.int32, (1, NBUCKET), 1)

    def g(b):
        return jnp.sum(jnp.where(iota >= b, cs, 0))

    def body(_, lh):
        lo, hi = lh
        mid = (lo + hi) // 2
        ge = g(mid) >= K
        return jnp.where(ge, mid, lo), jnp.where(ge, hi, mid)

    lo, _ = lax.fori_loop(
        0, 14, body, (jnp.int32(0), jnp.int32(NBUCKET))
    )
    nhi = g(lo + 1)
    row = lax.broadcasted_iota(jnp.int32, (8, 128), 0)
    col = lax.broadcasted_iota(jnp.int32, (8, 128), 1)
    val = jnp.where(col == 0, lo, jnp.where(col == 1, nhi, 0))
    out_ref[...] = jnp.where(row == 0, val, 0)


def _tc_findbucket(hist):
    return pl.pallas_call(
        _findbucket_body,
        out_shape=jax.ShapeDtypeStruct((8, 128), jnp.int32),
    )(hist)


@functools.partial(
    pl.kernel,
    out_type=(
        jax.ShapeDtypeStruct((NT, CAP), jnp.int32),
        jax.ShapeDtypeStruct((NT, CAP), jnp.int32),
    ),
    mesh=_MESH,
    compiler_params=pltpu.CompilerParams(needs_layout_passes=False),
    scratch_types=[
        pltpu.VMEM((N,), jnp.int32),
        pltpu.VMEM((N,), jnp.int32),
        pltpu.VMEM((CAP,), jnp.int32),
        pltpu.VMEM((CAP,), jnp.int32),
        pltpu.VMEM((16,), jnp.int32),
        pltpu.SemaphoreType.DMA,
        pltpu.SemaphoreType.DMA,
    ],
)
def _sc_compact(
    x_hbm, scal_hbm, kcand_out, icand_out, bufa, bufb, kbuf, ibuf, sbuf, sema, semb
):
    wid = lax.axis_index("s") * 2 + lax.axis_index("c")
    start = _row_start(wid)
    nrows = jnp.where(wid < 16, 313, 312)
    end = start + nrows
    pltpu.sync_copy(scal_hbm, sbuf)
    bstar = sbuf[...][0]

    neg1 = jnp.full((16,), -1, jnp.int32)

    def zbody(i, carry):
        for u in range(8):
            kbuf[pl.ds((i * 8 + u) * 16, 16)] = neg1
        return carry

    lax.fori_loop(0, CAP // 128, zbody, 0)

    lane = lax.iota(jnp.int32, 16)
    ones = jnp.ones((16,), jnp.int32)

    def issue(buf, sem, r):
        @pl.when(r < end)
        def _():
            pltpu.async_copy(x_hbm.at[r], buf, sem)

    def wait(buf, sem, r):
        pltpu.make_async_copy(x_hbm.at[r], buf, sem).wait()

    def process(buf, fb, curm1):
        def vb(i, curm1):
            off = i * 16
            kb = buf[pl.ds(off, 16)]
            msk = (kb >> NB_SHIFT) == bstar
            cnt = plsc.all_reduce_population_count(msk)
            pos = curm1 + plsc.cumsum(ones, mask=msk)
            okm = jnp.logical_and(msk, pos < CAP)
            plsc.store_scatter(kbuf, [pos], kb, mask=okm)
            plsc.store_scatter(ibuf, [pos], fb + off + lane, mask=okm)
            return curm1 + cnt

        return plsc.parallel_loop(0, N // 16, carry=curm1, unroll=12)(vb)

    issue(bufa, sema, start)
    issue(bufb, semb, start + 1)

    def pair_body(i, curm1):
        r = start + 2 * i
        wait(bufa, sema, r)
        curm1 = process(bufa, r * N, curm1)
        issue(bufa, sema, r + 2)
        wait(bufb, semb, r + 1)
        curm1 = process(bufb, (r + 1) * N, curm1)
        issue(bufb, semb, r + 3)
        return curm1

    curm1 = lax.fori_loop(0, nrows // 2, pair_body, jnp.full((16,), -1, jnp.int32))

    @pl.when(nrows % 2 == 1)
    def _():
        wait(bufa, sema, end - 1)
        process(bufa, (end - 1) * N, curm1)

    pltpu.sync_copy(kbuf, kcand_out.at[wid])
    pltpu.sync_copy(ibuf, icand_out.at[wid])


def _select_body(scal_ref, kc_ref, ic_ref, out_ref):
    bstar = scal_ref[0]
    kk = K - scal_ref[1]
    keys = kc_ref[...]
    idxs = ic_ref[...]
    lo0 = bstar << NB_SHIFT

    def cge(t):
        return jnp.sum((keys >= t).astype(jnp.int32))

    def b1(_, lh):
        lo, hi = lh
        mid = (lo + hi) // 2
        ge = cge(mid) >= kk
        return jnp.where(ge, mid, lo), jnp.where(ge, hi, mid)

    t, _ = lax.fori_loop(0, 16, b1, (lo0, lo0 + (1 << NB_SHIFT)))
    m = kk - cge(t + 1)
    eq = keys == t

    def feq(c):
        return jnp.sum(jnp.logical_and(eq, idxs <= c).astype(jnp.int32))

    def b2(_, lh):
        lo2, hi2 = lh
        mid = (lo2 + hi2) // 2
        ge = feq(mid) >= m
        return jnp.where(ge, lo2, mid), jnp.where(ge, mid, hi2)

    _, icut = lax.fori_loop(
        0, 27, b2, (jnp.int32(-1), jnp.int32(TOTAL - 1))
    )
    row = lax.broadcasted_iota(jnp.int32, (8, 128), 0)
    col = lax.broadcasted_iota(jnp.int32, (8, 128), 1)
    val = jnp.where(col == 0, t, jnp.where(col == 1, icut, 0))
    out_ref[...] = jnp.where(row == 0, val, 0)


def _tc_select(scal, kcand, icand):
    return pl.pallas_call(
        _select_body,
        in_specs=[
            pl.BlockSpec(memory_space=pltpu.SMEM),
            pl.BlockSpec(memory_space=pltpu.VMEM),
            pl.BlockSpec(memory_space=pltpu.VMEM),
        ],
        out_shape=jax.ShapeDtypeStruct((8, 128), jnp.int32),
    )(scal, kcand, icand)


BR = 80


def _emit_body(scal_ref, x_ref, w_ref, a_ref):
    t = scal_ref[0]
    icut = scal_ref[1]
    x = x_ref[...]
    kb = lax.bitcast_convert_type(x, jnp.int32)
    pid = pl.program_id(0)
    row = lax.broadcasted_iota(jnp.int32, (BR, N), 0)
    col = lax.broadcasted_iota(jnp.int32, (BR, N), 1)
    flat = (pid * BR + row) * N + col
    msk = (kb > t) | ((kb == t) & (flat <= icut))
    a_ref[...] = jnp.where(msk, jnp.float32(1.0), jnp.float32(0.0))
    w_ref[...] = jnp.where(msk, x, jnp.float32(0.0))


def _tc_emit(scal, x):
    grid = (N // BR,)
    blk = pl.BlockSpec((BR, N), lambda i: (i, 0))
    return pl.pallas_call(
        _emit_body,
        grid=grid,
        in_specs=[
            pl.BlockSpec(memory_space=pltpu.SMEM),
            blk,
        ],
        out_specs=(blk, blk),
        out_shape=(
            jax.ShapeDtypeStruct((N, N), jnp.float32),
            jax.ShapeDtypeStruct((N, N), jnp.float32),
        ),
    )(scal, x)


def kernel(weight_matrix):
    x = weight_matrix
    xi = lax.bitcast_convert_type(x, jnp.int32)
    hist = _sc_hist(xi)
    sc1 = _tc_findbucket(hist)
    scal16 = sc1[0, :16]
    kcand, icand = _sc_compact(xi, scal16)
    sc2 = _tc_select(sc1[0, :2], kcand, icand)
    wout, adj = _tc_emit(sc2[0, :2], x)
    return wout, adj


# speedup vs baseline: 1.0317x; 1.0317x over previous
import functools

import jax
import jax.numpy as jnp
from jax import lax
from jax.experimental import pallas as pl
from jax.experimental.pallas import tpu as pltpu
from jax.experimental.pallas import tpu_sc as plsc

N = 10000
TOTAL = N * N
K = N * 16
NB_SHIFT = 16
NBUCKET = 16384
CAP = 32768
NT = 32

_MESH = plsc.VectorSubcoreMesh(core_axis_name="c", subcore_axis_name="s")


def _row_start(wid):
    return jnp.where(wid < 16, wid * 313, 16 * 313 + (wid - 16) * 312)


@functools.partial(
    pl.kernel,
    out_type=jax.ShapeDtypeStruct((NT, NBUCKET), jnp.int32),
    mesh=_MESH,
    compiler_params=pltpu.CompilerParams(needs_layout_passes=False),
    scratch_types=[
        pltpu.VMEM((N,), jnp.int32),
        pltpu.VMEM((N,), jnp.int32),
        pltpu.VMEM((NBUCKET,), jnp.int32),
        pltpu.SemaphoreType.DMA,
        pltpu.SemaphoreType.DMA,
    ],
)
def _sc_hist(x_hbm, hist_out, bufa, bufb, histbuf, sema, semb):
    wid = lax.axis_index("s") * 2 + lax.axis_index("c")
    start = _row_start(wid)
    nrows = jnp.where(wid < 16, 313, 312)
    end = start + nrows
    zeros = jnp.zeros((16,), jnp.int32)

    def zbody(i, carry):
        for u in range(8):
            histbuf[pl.ds((i * 8 + u) * 16, 16)] = zeros
        return carry

    lax.fori_loop(0, NBUCKET // 128, zbody, 0)

    ones = jnp.ones((16,), jnp.int32)

    def issue(buf, sem, r):
        @pl.when(r < end)
        def _():
            pltpu.async_copy(x_hbm.at[r], buf, sem)

    def wait(buf, sem, r):
        pltpu.make_async_copy(x_hbm.at[r], buf, sem).wait()

    def process(buf):
        @plsc.parallel_loop(0, N // 16, unroll=8)
        def _(i):
            kb = buf[pl.ds(i * 16, 16)]
            plsc.addupdate_scatter(histbuf, [kb >> NB_SHIFT], ones)

    issue(bufa, sema, start)
    issue(bufb, semb, start + 1)

    def pair_body(i, carry):
        r = start + 2 * i
        wait(bufa, sema, r)
        process(bufa)
        issue(bufa, sema, r + 2)
        wait(bufb, semb, r + 1)
        process(bufb)
        issue(bufb, semb, r + 3)
        return carry

    lax.fori_loop(0, nrows // 2, pair_body, 0)

    @pl.when(nrows % 2 == 1)
    def _():
        wait(bufa, sema, end - 1)
        process(bufa)

    pltpu.sync_copy(histbuf, hist_out.at[wid])


def _findbucket_body(hist_ref, out_ref):
    h = hist_ref[...]
    cs = jnp.sum(h, axis=0, keepdims=True)
    iota = lax.broadcasted_iota(jnp.int32, (1, NBUCKET), 1)

    def g(b):
        return jnp.sum(jnp.where(iota >= b, cs, 0))

    def body(_, lh):
        lo, hi = lh
        mid = (lo + hi) // 2
        ge = g(mid) >= K
        return jnp.where(ge, mid, lo), jnp.where(ge, hi, mid)

    lo, _ = lax.fori_loop(
        0, 14, body, (jnp.int32(0), jnp.int32(NBUCKET))
    )
    nhi = g(lo + 1)
    row = lax.broadcasted_iota(jnp.int32, (8, 128), 0)
    col = lax.broadcasted_iota(jnp.int32, (8, 128), 1)
    val = jnp.where(col == 0, lo, jnp.where(col == 1, nhi, 0))
    out_ref[...] = jnp.where(row == 0, val, 0)


def _tc_findbucket(hist):
    return pl.pallas_call(
        _findbucket_body,
        out_shape=jax.ShapeDtypeStruct((8, 128), jnp.int32),
    )(hist)


@functools.partial(
    pl.kernel,
    out_type=(
        jax.ShapeDtypeStruct((NT, CAP), jnp.int32),
        jax.ShapeDtypeStruct((NT, CAP), jnp.int32),
    ),
    mesh=_MESH,
    compiler_params=pltpu.CompilerParams(needs_layout_passes=False),
    scratch_types=[
        pltpu.VMEM((N,), jnp.int32),
        pltpu.VMEM((N,), jnp.int32),
        pltpu.VMEM((CAP,), jnp.int32),
        pltpu.VMEM((CAP,), jnp.int32),
        pltpu.VMEM((16,), jnp.int32),
        pltpu.SemaphoreType.DMA,
        pltpu.SemaphoreType.DMA,
    ],
)
def _sc_compact(
    x_hbm, scal_hbm, kcand_out, icand_out, bufa, bufb, kbuf, ibuf, sbuf, sema, semb
):
    wid = lax.axis_index("s") * 2 + lax.axis_index("c")
    start = _row_start(wid)
    nrows = jnp.where(wid < 16, 313, 312)
    end = start + nrows
    pltpu.sync_copy(scal_hbm, sbuf)
    bstar = sbuf[...][0]

    neg1 = jnp.full((16,), -1, jnp.int32)

    def zbody(i, carry):
        for u in range(8):
            kbuf[pl.ds((i * 8 + u) * 16, 16)] = neg1
        return carry

    lax.fori_loop(0, CAP // 128, zbody, 0)

    lane = lax.iota(jnp.int32, 16)
    ones = jnp.ones((16,), jnp.int32)

    def issue(buf, sem, r):
        @pl.when(r < end)
        def _():
            pltpu.async_copy(x_hbm.at[r], buf, sem)

    def wait(buf, sem, r):
        pltpu.make_async_copy(x_hbm.at[r], buf, sem).wait()

    def process(buf, fb, curm1):
        def vb(i, curm1):
            off = i * 16
            kb = buf[pl.ds(off, 16)]
            msk = (kb >> NB_SHIFT) == bstar
            cnt = plsc.all_reduce_population_count(msk)
            pos = curm1 + plsc.cumsum(ones, mask=msk)
            okm = jnp.logical_and(msk, pos < CAP)
            plsc.store_scatter(kbuf, [pos], kb, mask=okm)
            plsc.store_scatter(ibuf, [pos], fb + off + lane, mask=okm)
            return curm1 + cnt

        return plsc.parallel_loop(0, N // 16, carry=curm1, unroll=8)(vb)

    issue(bufa, sema, start)
    issue(bufb, semb, start + 1)

    def pair_body(i, curm1):
        r = start + 2 * i
        wait(bufa, sema, r)
        curm1 = process(bufa, r * N, curm1)
        issue(bufa, sema, r + 2)
        wait(bufb, semb, r + 1)
        curm1 = process(bufb, (r + 1) * N, curm1)
        issue(bufb, semb, r + 3)
        return curm1

    curm1 = lax.fori_loop(0, nrows // 2, pair_body, jnp.full((16,), -1, jnp.int32))

    @pl.when(nrows % 2 == 1)
    def _():
        wait(bufa, sema, end - 1)
        process(bufa, (end - 1) * N, curm1)

    pltpu.sync_copy(kbuf, kcand_out.at[wid])
    pltpu.sync_copy(ibuf, icand_out.at[wid])


def _select_body(scal_ref, kc_ref, ic_ref, out_ref):
    bstar = scal_ref[0]
    kk = K - scal_ref[1]
    keys = kc_ref[...]
    idxs = ic_ref[...]
    lo0 = bstar << NB_SHIFT

    def cge(t):
        return jnp.sum((keys >= t).astype(jnp.int32))

    def b1(_, lh):
        lo, hi = lh
        mid = (lo + hi) // 2
        ge = cge(mid) >= kk
        return jnp.where(ge, mid, lo), jnp.where(ge, hi, mid)

    t, _ = lax.fori_loop(0, 16, b1, (lo0, lo0 + (1 << NB_SHIFT)))
    m = kk - cge(t + 1)
    eq = keys == t

    def feq(c):
        return jnp.sum(jnp.logical_and(eq, idxs <= c).astype(jnp.int32))

    def b2(_, lh):
        lo2, hi2 = lh
        mid = (lo2 + hi2) // 2
        ge = feq(mid) >= m
        return jnp.where(ge, lo2, mid), jnp.where(ge, mid, hi2)

    _, icut = lax.fori_loop(
        0, 27, b2, (jnp.int32(-1), jnp.int32(TOTAL - 1))
    )
    row = lax.broadcasted_iota(jnp.int32, (8, 128), 0)
    col = lax.broadcasted_iota(jnp.int32, (8, 128), 1)
    val = jnp.where(col == 0, t, jnp.where(col == 1, icut, 0))
    out_ref[...] = jnp.where(row == 0, val, 0)


def _tc_select(scal, kcand, icand):
    return pl.pallas_call(
        _select_body,
        in_specs=[
            pl.BlockSpec(memory_space=pltpu.SMEM),
            pl.BlockSpec(memory_space=pltpu.VMEM),
            pl.BlockSpec(memory_space=pltpu.VMEM),
        ],
        out_shape=jax.ShapeDtypeStruct((8, 128), jnp.int32),
    )(scal, kcand, icand)


BR = 80


def _emit_body(scal_ref, x_ref, w_ref, a_ref):
    t = scal_ref[0]
    icut = scal_ref[1]
    x = x_ref[...]
    kb = lax.bitcast_convert_type(x, jnp.int32)
    pid = pl.program_id(0)
    row = lax.broadcasted_iota(jnp.int32, (BR, N), 0)
    col = lax.broadcasted_iota(jnp.int32, (BR, N), 1)
    flat = (pid * BR + row) * N + col
    msk = (kb > t) | ((kb == t) & (flat <= icut))
    a_ref[...] = jnp.where(msk, jnp.float32(1.0), jnp.float32(0.0))
    w_ref[...] = jnp.where(msk, x, jnp.float32(0.0))


def _tc_emit(scal, x):
    grid = (N // BR,)
    blk = pl.BlockSpec((BR, N), lambda i: (i, 0))
    return pl.pallas_call(
        _emit_body,
        grid=grid,
        in_specs=[
            pl.BlockSpec(memory_space=pltpu.SMEM),
            blk,
        ],
        out_specs=(blk, blk),
        out_shape=(
            jax.ShapeDtypeStruct((N, N), jnp.float32),
            jax.ShapeDtypeStruct((N, N), jnp.float32),
        ),
    )(scal, x)


def kernel(weight_matrix):
    x = weight_matrix
    xi = lax.bitcast_convert_type(x, jnp.int32)
    hist = _sc_hist(xi)
    sc1 = _tc_findbucket(hist)
    scal16 = sc1[0, :16]
    kcand, icand = _sc_compact(xi, scal16)
    sc2 = _tc_select(sc1[0, :2], kcand, icand)
    wout, adj = _tc_emit(sc2[0, :2], x)
    return wout, adj
